# Initial kernel scaffold; baseline (speedup 1.0000x reference)
import functools

import jax
import jax.numpy as jnp
from jax import lax
from jax.experimental import pallas as pl
from jax.experimental.pallas import tpu as pltpu
from jax.experimental.pallas import tpu_sc as plsc

N = 10000
G = 16
D = 128
D_HID = 256
D_OUT = 64

N_PAD = 10240
E_PAD = 327680
CH = 128
NC, NS = 2, 16
NW = NC * NS
CPT = E_PAD // (NW * CH)
GRP = 5
NGRP = CPT // GRP
RPT = N_PAD // NS

BLK = 2048
F32 = jnp.float32

_MESH = plsc.VectorSubcoreMesh(
    core_axis_name="c", subcore_axis_name="s", num_cores=NC, num_subcores=NS)


def _sc_deg_body(didx_hbm, ones_hbm, zeros_hbm, out_hbm, didx_v, ones_v, acc,
                 sem):
    c = lax.axis_index("c")
    s = lax.axis_index("s")
    wid = c * NS + s
    pltpu.sync_copy(zeros_hbm.at[pl.ds(s * RPT, RPT)],
                    acc.at[pl.ds(s * RPT, RPT)])
    pltpu.sync_copy(didx_hbm.at[pl.ds(wid * CPT, CPT)], didx_v)
    pltpu.sync_copy(ones_hbm, ones_v)
    plsc.subcore_barrier()

    def grp(gi, carry):
        base = gi * GRP
        descs = [
            pltpu.async_copy(ones_v, acc.at[didx_v.at[base + k]], sem,
                             add=True)
            for k in range(GRP)
        ]
        for d in descs:
            d.wait()
        return carry

    lax.fori_loop(0, NGRP, grp, 0)
    plsc.subcore_barrier()
    pltpu.sync_copy(acc.at[pl.ds(s * RPT, RPT)],
                    out_hbm.at[c, pl.ds(s * RPT, RPT)])


_sc_deg = functools.partial(
    pl.kernel,
    out_type=jax.ShapeDtypeStruct((NC, N_PAD, 16), F32),
    mesh=_MESH,
    scratch_types=[
        pltpu.VMEM((CPT, CH), jnp.int32),
        pltpu.VMEM((CH, 16), F32),
        pltpu.VMEM_SHARED((N_PAD, 16), F32),
        pltpu.SemaphoreType.DMA,
    ],
)(_sc_deg_body)


def _sc_scatter_body(g_hbm, sidx_hbm, didx_hbm, zeros_hbm, out_hbm,
                     sidx_v, didx_v, rows_v, acc, sem):
    c = lax.axis_index("c")
    s = lax.axis_index("s")
    wid = c * NS + s
    pltpu.sync_copy(zeros_hbm.at[pl.ds(s * RPT, RPT)],
                    acc.at[pl.ds(s * RPT, RPT)])
    pltpu.sync_copy(sidx_hbm.at[pl.ds(wid * CPT, CPT)], sidx_v)
    pltpu.sync_copy(didx_hbm.at[pl.ds(wid * CPT, CPT)], didx_v)
    plsc.subcore_barrier()

    def grp(gi, carry):
        base = gi * GRP
        descs = [
            pltpu.async_copy(g_hbm.at[sidx_v.at[base + k]],
                             rows_v.at[pl.ds(k * CH, CH)], sem)
            for k in range(GRP)
        ]
        for d in descs:
            d.wait()
        for k in range(GRP):
            pltpu.sync_copy(rows_v.at[pl.ds(k * CH, CH)],
                            acc.at[didx_v.at[base + k]], add=True)
        return carry

    lax.fori_loop(0, NGRP, grp, 0)
    plsc.subcore_barrier()
    pltpu.sync_copy(acc.at[pl.ds(s * RPT, RPT)],
                    out_hbm.at[c, pl.ds(s * RPT, RPT)])


_sc_scatter = functools.partial(
    pl.kernel,
    out_type=jax.ShapeDtypeStruct((NC, N_PAD, D), F32),
    mesh=_MESH,
    scratch_types=[
        pltpu.VMEM((CPT, CH), jnp.int32),
        pltpu.VMEM((CPT, CH), jnp.int32),
        pltpu.VMEM((GRP * CH, D), F32),
        pltpu.VMEM_SHARED((N_PAD, D), F32),
        pltpu.SemaphoreType.DMA,
    ],
)(_sc_scatter_body)


def _tc_prep_body(x_ref, degp_ref, w_ref, g_ref, dis_ref):
    degp = degp_ref[...]
    deg = degp[0, :, 0:1] + degp[1, :, 0:1] + 1.0
    dis = jnp.broadcast_to(lax.rsqrt(deg), (BLK, D))
    dis_ref[...] = dis
    g_ref[...] = dis * jnp.dot(x_ref[...], w_ref[...],
                               preferred_element_type=F32)


_tc_prep = pl.pallas_call(
    _tc_prep_body,
    grid=(N_PAD // BLK,),
    in_specs=[
        pl.BlockSpec((BLK, D), lambda i: (i, 0)),
        pl.BlockSpec((NC, BLK, 16), lambda i: (0, i, 0)),
        pl.BlockSpec((D, D), lambda i: (0, 0)),
    ],
    out_specs=[
        pl.BlockSpec((BLK, D), lambda i: (i, 0)),
        pl.BlockSpec((BLK, D), lambda i: (i, 0)),
    ],
    out_shape=[
        jax.ShapeDtypeStruct((N_PAD, D), F32),
        jax.ShapeDtypeStruct((N_PAD, D), F32),
    ],
)


def _tc_mid_body(p_ref, gprev_ref, dis_ref, b_ref, w_ref, gnext_ref):
    p = p_ref[...]
    dis = dis_ref[...]
    acc = p[0] + p[1] + gprev_ref[...]
    t = jnp.maximum(dis * acc + b_ref[...], 0.0)
    gnext_ref[...] = dis * jnp.dot(t, w_ref[...], preferred_element_type=F32)


_tc_mid = pl.pallas_call(
    _tc_mid_body,
    grid=(N_PAD // BLK,),
    in_specs=[
        pl.BlockSpec((NC, BLK, D), lambda i: (0, i, 0)),
        pl.BlockSpec((BLK, D), lambda i: (i, 0)),
        pl.BlockSpec((BLK, D), lambda i: (i, 0)),
        pl.BlockSpec((1, D), lambda i: (0, 0)),
        pl.BlockSpec((D, D), lambda i: (0, 0)),
    ],
    out_specs=pl.BlockSpec((BLK, D), lambda i: (i, 0)),
    out_shape=jax.ShapeDtypeStruct((N_PAD, D), F32),
)


def _tc_final_body(p_ref, g2_ref, dis_ref, b_ref, batch_ref,
                   wd0_ref, bd0_ref, wd1_ref, bd1_ref, out_ref,
                   sums_ref, cnts_ref):
    i = pl.program_id(0)

    @pl.when(i == 0)
    def _():
        sums_ref[...] = jnp.zeros_like(sums_ref)
        cnts_ref[...] = jnp.zeros_like(cnts_ref)

    p = p_ref[...]
    h3 = jnp.maximum(dis_ref[...] * (p[0] + p[1] + g2_ref[...]) + b_ref[...],
                     0.0)
    grp_ids = lax.broadcasted_iota(jnp.int32, (BLK, G), 1)
    oh = (batch_ref[...] == grp_ids).astype(F32)
    dims = (((0,), (0,)), ((), ()))
    sums_ref[...] += lax.dot_general(oh, h3, dims,
                                     preferred_element_type=F32)
    cnts_ref[...] += lax.dot_general(oh, jnp.ones((BLK, D), F32), dims,
                                     preferred_element_type=F32)

    @pl.when(i == pl.num_programs(0) - 1)
    def _():
        pooled = sums_ref[...] / jnp.maximum(cnts_ref[...], 1.0)
        h = jnp.maximum(
            jnp.dot(pooled, wd0_ref[...], preferred_element_type=F32)
            + bd0_ref[...], 0.0)
        out_ref[...] = (jnp.dot(h, wd1_ref[...], preferred_element_type=F32)
                        + bd1_ref[...])


_tc_final = pl.pallas_call(
    _tc_final_body,
    grid=(N_PAD // BLK,),
    in_specs=[
        pl.BlockSpec((NC, BLK, D), lambda i: (0, i, 0)),
        pl.BlockSpec((BLK, D), lambda i: (i, 0)),
        pl.BlockSpec((BLK, D), lambda i: (i, 0)),
        pl.BlockSpec((1, D), lambda i: (0, 0)),
        pl.BlockSpec((BLK, 1), lambda i: (i, 0)),
        pl.BlockSpec((D, D_HID), lambda i: (0, 0)),
        pl.BlockSpec((1, D_HID), lambda i: (0, 0)),
        pl.BlockSpec((D_HID, D_OUT), lambda i: (0, 0)),
        pl.BlockSpec((1, D_OUT), lambda i: (0, 0)),
    ],
    out_specs=pl.BlockSpec((G, D_OUT), lambda i: (0, 0)),
    out_shape=jax.ShapeDtypeStruct((G, D_OUT), F32),
    scratch_shapes=[
        pltpu.VMEM((G, D), F32),
        pltpu.VMEM((G, D), F32),
    ],
)


def kernel(x, edge_index, batch, Wc0, bc0, Wc1, bc1, Wc2, bc2,
           Wd0, bd0, Wd1, bd1):
    src = edge_index[0].astype(jnp.int32)
    dst = edge_index[1].astype(jnp.int32)
    e = src.shape[0]
    dummy = jnp.full((E_PAD - e,), N, jnp.int32)
    sidx = jnp.concatenate([src, dummy]).reshape(NW * CPT, CH)
    didx = jnp.concatenate([dst, dummy]).reshape(NW * CPT, CH)
    x_pad = jnp.concatenate(
        [x.astype(F32), jnp.zeros((N_PAD - N, D), F32)])
    batch2d = jnp.concatenate(
        [batch.astype(jnp.int32),
         jnp.full((N_PAD - N,), G, jnp.int32)]).reshape(N_PAD, 1)
    zeros128 = jnp.zeros((N_PAD, D), F32)
    zeros16 = jnp.zeros((N_PAD, 16), F32)
    ones16 = jnp.ones((CH, 16), F32)

    degp = _sc_deg(didx, ones16, zeros16)
    g0, dis = _tc_prep(x_pad, degp, Wc0)
    p = _sc_scatter(g0, sidx, didx, zeros128)
    g1 = _tc_mid(p, g0, dis, bc0.reshape(1, D), Wc1)
    p = _sc_scatter(g1, sidx, didx, zeros128)
    g2 = _tc_mid(p, g1, dis, bc1.reshape(1, D), Wc2)
    p = _sc_scatter(g2, sidx, didx, zeros128)
    out = _tc_final(p, g2, dis, bc2.reshape(1, D), batch2d,
                    Wd0, bd0.reshape(1, D_HID), Wd1, bd1.reshape(1, D_OUT))
    return out


# speedup vs baseline: 9.5704x; 9.5704x over previous
import functools

import jax
import jax.numpy as jnp
from jax import lax
from jax.experimental import pallas as pl
from jax.experimental.pallas import tpu as pltpu
from jax.experimental.pallas import tpu_sc as plsc

N = 10000
G = 16
D = 128
D_HID = 256
D_OUT = 64

N_PAD = 10240
E_PAD = 327680
CH = 128
NC, NS = 2, 16
NW = NC * NS
NCHUNK = E_PAD // CH
CPT = NCHUNK // NW
PAIRS = CPT // 2
DEG_GRP = 8
RPT = N_PAD // NS

BLK = 2048
F32 = jnp.float32

_MESH = plsc.VectorSubcoreMesh(
    core_axis_name="c", subcore_axis_name="s", num_cores=NC, num_subcores=NS)


def _sc_deg_body(didx_hbm, ones_hbm, zeros_hbm, out_hbm, didx_v, ones_v, acc,
                 sem):
    c = lax.axis_index("c")
    s = lax.axis_index("s")
    wid = c * NS + s
    pltpu.sync_copy(zeros_hbm.at[pl.ds(s * RPT, RPT)],
                    acc.at[pl.ds(s * RPT, RPT)])
    pltpu.sync_copy(didx_hbm.at[pl.ds(wid * CPT, CPT)], didx_v)
    pltpu.sync_copy(ones_hbm, ones_v)
    plsc.subcore_barrier()

    def grp(gi, carry):
        base = gi * DEG_GRP
        descs = [
            pltpu.async_copy(ones_v, acc.at[didx_v.at[base + k]], sem,
                             add=True)
            for k in range(DEG_GRP)
        ]
        for d in descs:
            d.wait()
        return carry

    lax.fori_loop(0, CPT // DEG_GRP, grp, 0)
    plsc.subcore_barrier()
    pltpu.sync_copy(acc.at[pl.ds(s * RPT, RPT)],
                    out_hbm.at[c, pl.ds(s * RPT, RPT)])


_sc_deg = functools.partial(
    pl.kernel,
    out_type=jax.ShapeDtypeStruct((NC, N_PAD, 16), F32),
    mesh=_MESH,
    scratch_types=[
        pltpu.VMEM((CPT, CH), jnp.int32),
        pltpu.VMEM((CH, 16), F32),
        pltpu.VMEM_SHARED((N_PAD, 16), F32),
        pltpu.SemaphoreType.DMA,
    ],
)(_sc_deg_body)


def _sc_scatter_body(g_hbm, sidx_hbm, didx_hbm, zeros_hbm, out_hbm,
                     sidx_v, dring, buf0, buf1, acc, semA, semB, semI):
    c = lax.axis_index("c")
    s = lax.axis_index("s")
    wid = c * NS + s
    base = wid * CPT

    pltpu.sync_copy(zeros_hbm.at[pl.ds(s * RPT, RPT)],
                    acc.at[pl.ds(s * RPT, RPT)])
    pltpu.sync_copy(sidx_hbm.at[pl.ds(base, CPT)], sidx_v)
    pltpu.sync_copy(didx_hbm.at[pl.ds(base, 2)], dring.at[pl.ds(0, 2)])
    plsc.subcore_barrier()

    pltpu.async_copy(g_hbm.at[sidx_v.at[0]], buf0, semA)
    pltpu.async_copy(g_hbm.at[sidx_v.at[1]], buf1, semB)
    pltpu.async_copy(didx_hbm.at[base + 2], dring.at[2], semI)
    pltpu.async_copy(didx_hbm.at[base + 3], dring.at[3], semI)

    def pair(j, carry):
        c0 = 2 * j
        pltpu.make_async_copy(didx_hbm.at[base], dring.at[0], semI).wait()
        pltpu.make_async_copy(didx_hbm.at[base], dring.at[0], semI).wait()
        pltpu.make_async_copy(g_hbm.at[sidx_v.at[c0]], buf0, semA).wait()
        pltpu.sync_copy(buf0, acc.at[dring.at[lax.rem(c0, 4)]], add=True)
        pltpu.async_copy(g_hbm.at[sidx_v.at[c0 + 2]], buf0, semA)
        pltpu.async_copy(didx_hbm.at[base + c0 + 4],
                         dring.at[lax.rem(c0, 4)], semI)
        c1 = c0 + 1
        pltpu.make_async_copy(g_hbm.at[sidx_v.at[c1]], buf1, semB).wait()
        pltpu.sync_copy(buf1, acc.at[dring.at[lax.rem(c1, 4)]], add=True)
        pltpu.async_copy(g_hbm.at[sidx_v.at[c1 + 2]], buf1, semB)
        pltpu.async_copy(didx_hbm.at[base + c1 + 4],
                         dring.at[lax.rem(c1, 4)], semI)
        return carry

    lax.fori_loop(0, PAIRS - 1, pair, 0)

    pltpu.make_async_copy(didx_hbm.at[base], dring.at[0], semI).wait()
    pltpu.make_async_copy(didx_hbm.at[base], dring.at[0], semI).wait()
    last = CPT - 2
    pltpu.make_async_copy(g_hbm.at[sidx_v.at[last]], buf0, semA).wait()
    pltpu.sync_copy(buf0, acc.at[dring.at[lax.rem(last, 4)]], add=True)
    pltpu.make_async_copy(g_hbm.at[sidx_v.at[last + 1]], buf1, semB).wait()
    pltpu.sync_copy(buf1, acc.at[dring.at[lax.rem(last + 1, 4)]], add=True)

    plsc.subcore_barrier()
    pltpu.sync_copy(acc.at[pl.ds(s * RPT, RPT)],
                    out_hbm.at[c, pl.ds(s * RPT, RPT)])


_sc_scatter = functools.partial(
    pl.kernel,
    out_type=jax.ShapeDtypeStruct((NC, N_PAD, D), F32),
    mesh=_MESH,
    scratch_types=[
        pltpu.VMEM((CPT, CH), jnp.int32),
        pltpu.VMEM((4, CH), jnp.int32),
        pltpu.VMEM((CH, D), F32),
        pltpu.VMEM((CH, D), F32),
        pltpu.VMEM_SHARED((N_PAD, D), F32),
        pltpu.SemaphoreType.DMA,
        pltpu.SemaphoreType.DMA,
        pltpu.SemaphoreType.DMA,
    ],
)(_sc_scatter_body)


def _tc_prep_body(x_ref, degp_ref, w_ref, g_ref, dis_ref):
    degp = degp_ref[...]
    deg = degp[0, :, 0:1] + degp[1, :, 0:1] + 1.0
    dis = jnp.broadcast_to(lax.rsqrt(deg), (BLK, D))
    dis_ref[...] = dis
    g_ref[...] = dis * jnp.dot(x_ref[...], w_ref[...],
                               preferred_element_type=F32)


_tc_prep = pl.pallas_call(
    _tc_prep_body,
    grid=(N_PAD // BLK,),
    in_specs=[
        pl.BlockSpec((BLK, D), lambda i: (i, 0)),
        pl.BlockSpec((NC, BLK, 16), lambda i: (0, i, 0)),
        pl.BlockSpec((D, D), lambda i: (0, 0)),
    ],
    out_specs=[
        pl.BlockSpec((BLK, D), lambda i: (i, 0)),
        pl.BlockSpec((BLK, D), lambda i: (i, 0)),
    ],
    out_shape=[
        jax.ShapeDtypeStruct((N_PAD, D), F32),
        jax.ShapeDtypeStruct((N_PAD, D), F32),
    ],
)


def _tc_mid_body(p_ref, gprev_ref, dis_ref, b_ref, w_ref, gnext_ref):
    p = p_ref[...]
    dis = dis_ref[...]
    acc = p[0] + p[1] + gprev_ref[...]
    t = jnp.maximum(dis * acc + b_ref[...], 0.0)
    gnext_ref[...] = dis * jnp.dot(t, w_ref[...], preferred_element_type=F32)


_tc_mid = pl.pallas_call(
    _tc_mid_body,
    grid=(N_PAD // BLK,),
    in_specs=[
        pl.BlockSpec((NC, BLK, D), lambda i: (0, i, 0)),
        pl.BlockSpec((BLK, D), lambda i: (i, 0)),
        pl.BlockSpec((BLK, D), lambda i: (i, 0)),
        pl.BlockSpec((1, D), lambda i: (0, 0)),
        pl.BlockSpec((D, D), lambda i: (0, 0)),
    ],
    out_specs=pl.BlockSpec((BLK, D), lambda i: (i, 0)),
    out_shape=jax.ShapeDtypeStruct((N_PAD, D), F32),
)


def _tc_final_body(p_ref, g2_ref, dis_ref, b_ref, batch_ref,
                   wd0_ref, bd0_ref, wd1_ref, bd1_ref, out_ref,
                   sums_ref, cnts_ref):
    i = pl.program_id(0)

    @pl.when(i == 0)
    def _():
        sums_ref[...] = jnp.zeros_like(sums_ref)
        cnts_ref[...] = jnp.zeros_like(cnts_ref)

    p = p_ref[...]
    h3 = jnp.maximum(dis_ref[...] * (p[0] + p[1] + g2_ref[...]) + b_ref[...],
                     0.0)
    grp_ids = lax.broadcasted_iota(jnp.int32, (BLK, G), 1)
    oh = (batch_ref[...] == grp_ids).astype(F32)
    dims = (((0,), (0,)), ((), ()))
    sums_ref[...] += lax.dot_general(oh, h3, dims,
                                     preferred_element_type=F32)
    cnts_ref[...] += lax.dot_general(oh, jnp.ones((BLK, D), F32), dims,
                                     preferred_element_type=F32)

    @pl.when(i == pl.num_programs(0) - 1)
    def _():
        pooled = sums_ref[...] / jnp.maximum(cnts_ref[...], 1.0)
        h = jnp.maximum(
            jnp.dot(pooled, wd0_ref[...], preferred_element_type=F32)
            + bd0_ref[...], 0.0)
        out_ref[...] = (jnp.dot(h, wd1_ref[...], preferred_element_type=F32)
                        + bd1_ref[...])


_tc_final = pl.pallas_call(
    _tc_final_body,
    grid=(N_PAD // BLK,),
    in_specs=[
        pl.BlockSpec((NC, BLK, D), lambda i: (0, i, 0)),
        pl.BlockSpec((BLK, D), lambda i: (i, 0)),
        pl.BlockSpec((BLK, D), lambda i: (i, 0)),
        pl.BlockSpec((1, D), lambda i: (0, 0)),
        pl.BlockSpec((BLK, 1), lambda i: (i, 0)),
        pl.BlockSpec((D, D_HID), lambda i: (0, 0)),
        pl.BlockSpec((1, D_HID), lambda i: (0, 0)),
        pl.BlockSpec((D_HID, D_OUT), lambda i: (0, 0)),
        pl.BlockSpec((1, D_OUT), lambda i: (0, 0)),
    ],
    out_specs=pl.BlockSpec((G, D_OUT), lambda i: (0, 0)),
    out_shape=jax.ShapeDtypeStruct((G, D_OUT), F32),
    scratch_shapes=[
        pltpu.VMEM((G, D), F32),
        pltpu.VMEM((G, D), F32),
    ],
)


def kernel(x, edge_index, batch, Wc0, bc0, Wc1, bc1, Wc2, bc2,
           Wd0, bd0, Wd1, bd1):
    src = edge_index[0].astype(jnp.int32)
    dst = edge_index[1].astype(jnp.int32)
    e = src.shape[0]
    dummy = jnp.full((E_PAD - e,), N, jnp.int32)
    sidx = jnp.concatenate([src, dummy]).reshape(NCHUNK, CH)
    didx = jnp.concatenate(
        [dst, dummy, jnp.full((2 * CH,), N, jnp.int32)]).reshape(
            NCHUNK + 2, CH)
    x_pad = jnp.concatenate(
        [x.astype(F32), jnp.zeros((N_PAD - N, D), F32)])
    batch2d = jnp.concatenate(
        [batch.astype(jnp.int32),
         jnp.full((N_PAD - N,), G, jnp.int32)]).reshape(N_PAD, 1)
    zeros128 = jnp.zeros((N_PAD, D), F32)
    zeros16 = jnp.zeros((N_PAD, 16), F32)
    ones16 = jnp.ones((CH, 16), F32)

    degp = _sc_deg(didx, ones16, zeros16)
    g0, dis = _tc_prep(x_pad, degp, Wc0)
    p = _sc_scatter(g0, sidx, didx, zeros128)
    g1 = _tc_mid(p, g0, dis, bc0.reshape(1, D), Wc1)
    p = _sc_scatter(g1, sidx, didx, zeros128)
    g2 = _tc_mid(p, g1, dis, bc1.reshape(1, D), Wc2)
    p = _sc_scatter(g2, sidx, didx, zeros128)
    out = _tc_final(p, g2, dis, bc2.reshape(1, D), batch2d,
                    Wd0, bd0.reshape(1, D_HID), Wd1, bd1.reshape(1, D_OUT))
    return out
